# Initial kernel scaffold; baseline (speedup 1.0000x reference)
import jax
import jax.numpy as jnp
from jax.experimental import pallas as pl


def kernel(images, features, w_conv, b_conv, w_cls, b_cls, w_bbox, b_bbox):
    raise NotImplementedError("write your pallas kernel here")


# speedup vs baseline: 12.2503x; 12.2503x over previous
import math

import numpy as np
import jax
import jax.numpy as jnp
from jax.experimental import pallas as pl
from jax.experimental.pallas import tpu as pltpu

SCALES = (8., 16., 32., 64., 128., 256., 512., 1024., 2048., 4096.)
RATIO = 0.125
NMS_THR = 0.7
MIN_SIZE = 1e-3
BBOX_CLIP = math.log(1000.0 / 16.0)


def _cell_anchors_np():
    scales = np.array(SCALES, dtype=np.float32)
    h_r = np.sqrt(np.array([RATIO], dtype=np.float32))
    w_r = (np.float32(1.0) / h_r).astype(np.float32)
    ws = (w_r[:, None] * scales[None, :]).reshape(-1).astype(np.float32)
    hs = (h_r[:, None] * scales[None, :]).reshape(-1).astype(np.float32)
    return np.round(np.stack([-ws, -hs, ws, hs], axis=1).astype(np.float32) / 2.0)

_CELL = _cell_anchors_np()


def _conv_kernel(xpad_ref, wtaps_ref, bconv_ref, whead_ref, bhead_ref, out_ref):
    lane = jax.lax.broadcasted_iota(jnp.int32, (4096, 1), 0) % 64
    acc = jnp.zeros((4096, 256), jnp.float32)
    for k in range(9):
        dy, dx = k // 3 - 1, k % 3 - 1
        off = 72 + dy * 64 + dx
        sl = xpad_ref[0, pl.ds(off, 4096), :]
        if dx == -1:
            sl = jnp.where(lane == 0, 0.0, sl)
        elif dx == 1:
            sl = jnp.where(lane == 63, 0.0, sl)
        acc = acc + jnp.dot(sl, wtaps_ref[k],
                            preferred_element_type=jnp.float32)
    t = jnp.maximum(acc + bconv_ref[0][None, :], 0.0)
    out_ref[0] = (jnp.dot(t, whead_ref[...], preferred_element_type=jnp.float32)
                  + bhead_ref[0][None, :])


def _conv_heads(features, w_conv, b_conv, w_cls, b_cls, w_bbox, b_bbox):
    B = features.shape[0]
    x = jnp.transpose(features, (0, 2, 3, 1)).reshape(B, 4096, 256)
    xpad = jnp.pad(x, ((0, 0), (72, 72), (0, 0)))
    wtaps = jnp.transpose(w_conv, (2, 3, 1, 0)).reshape(9, 256, 256)
    whead = jnp.concatenate([w_cls[:, :, 0, 0], w_bbox[:, :, 0, 0]], axis=0).T
    whead = jnp.pad(whead, ((0, 0), (0, 14)))
    bhead = jnp.pad(jnp.concatenate([b_cls, b_bbox]), (0, 14))[None, :]
    return pl.pallas_call(
        _conv_kernel,
        grid=(B,),
        in_specs=[
            pl.BlockSpec((1, 4240, 256), lambda i: (i, 0, 0)),
            pl.BlockSpec((9, 256, 256), lambda i: (0, 0, 0)),
            pl.BlockSpec((1, 256), lambda i: (0, 0)),
            pl.BlockSpec((256, 64), lambda i: (0, 0)),
            pl.BlockSpec((1, 64), lambda i: (0, 0)),
        ],
        out_specs=pl.BlockSpec((1, 4096, 64), lambda i: (i, 0, 0)),
        out_shape=jax.ShapeDtypeStruct((B, 4096, 64), jnp.float32),
    )(xpad, wtaps, b_conv[None, :], whead, bhead)


def _comp_gt(av, ai, bv, bi):
    return (av > bv) | ((av == bv) & (ai < bi))


def _flat_idx(shape):
    s = jax.lax.broadcasted_iota(jnp.int32, shape, len(shape) - 2)
    l = jax.lax.broadcasted_iota(jnp.int32, shape, len(shape) - 1)
    return s * 128 + l


def _xor_perm(x, j):
    if j < 128:
        l = jax.lax.broadcasted_iota(jnp.int32, x.shape, x.ndim - 1)
        n = x.shape[x.ndim - 1]
        lo = pltpu.roll(x, n - j, x.ndim - 1)
        hi = pltpu.roll(x, j, x.ndim - 1)
        return jnp.where((l & j) == 0, lo, hi)
    js = j // 128
    s = jax.lax.broadcasted_iota(jnp.int32, x.shape, x.ndim - 2)
    n = x.shape[x.ndim - 2]
    lo = pltpu.roll(x, n - js, x.ndim - 2)
    hi = pltpu.roll(x, js, x.ndim - 2)
    return jnp.where((s & js) == 0, lo, hi)


def _stage(v, ix, j, want_larger):
    pv = _xor_perm(v, j)
    pix = _xor_perm(ix, j)
    self_larger = _comp_gt(v, ix, pv, pix)
    sel = want_larger == self_larger
    return jnp.where(sel, v, pv), jnp.where(sel, ix, pix)


def _chunk_odd(shape):
    return (jax.lax.broadcasted_iota(jnp.int32, shape, 1) & 1) == 1


def _topk_kernel(s_ref, v_ref, ix_ref):
    v = s_ref[...]
    B = v.shape[0]
    ix = _flat_idx(v.shape) + 1024 * jax.lax.broadcasted_iota(
        jnp.int32, v.shape, 1)
    i = _flat_idx(v.shape)
    odd = _chunk_odd(v.shape)
    k = 2
    while k <= 1024:
        j = k // 2
        while j >= 1:
            wl = (((i & k) == 0) == ((i & j) == 0)) ^ odd
            v, ix = _stage(v, ix, j, wl)
            j //= 2
        k *= 2
    v = jnp.concatenate(
        [v, jnp.full((B, 24, 8, 128), -jnp.inf, jnp.float32)], axis=1)
    ix = jnp.concatenate([ix, jnp.zeros((B, 24, 8, 128), jnp.int32)], axis=1)
    m = 64
    while m > 1:
        v = v.reshape(B, m // 2, 2, 8, 128)
        ix = ix.reshape(B, m // 2, 2, 8, 128)
        av, bv, ai, bi = v[:, :, 0], v[:, :, 1], ix[:, :, 0], ix[:, :, 1]
        take = _comp_gt(av, ai, bv, bi)
        v = jnp.where(take, av, bv)
        ix = jnp.where(take, ai, bi)
        i = _flat_idx(v.shape)
        odd = _chunk_odd(v.shape)
        j = 512
        while j >= 1:
            wl = ((i & j) == 0) ^ odd
            v, ix = _stage(v, ix, j, wl)
            j //= 2
        m //= 2
    v_ref[...] = v[:, 0]
    ix_ref[...] = ix[:, 0]


def _topk1024(scores):
    B = scores.shape[0]
    return pl.pallas_call(
        _topk_kernel,
        out_shape=[jax.ShapeDtypeStruct((B, 8, 128), jnp.float32),
                   jax.ShapeDtypeStruct((B, 8, 128), jnp.int32)],
    )(scores.reshape(B, 40, 8, 128))


def _decode_frame(idx, d0, d1, d2, d3, vals, rank):
    a = idx % 10
    pix = idx // 10
    gy = (pix // 64).astype(jnp.float32) * 8.0
    gx = (pix % 64).astype(jnp.float32) * 8.0
    c0 = jnp.zeros_like(gx)
    c1 = jnp.zeros_like(gx)
    c2 = jnp.zeros_like(gx)
    c3 = jnp.zeros_like(gx)
    for av in range(10):
        m = a == av
        c0 = jnp.where(m, float(_CELL[av, 0]), c0)
        c1 = jnp.where(m, float(_CELL[av, 1]), c1)
        c2 = jnp.where(m, float(_CELL[av, 2]), c2)
        c3 = jnp.where(m, float(_CELL[av, 3]), c3)
    a0, a1, a2, a3 = gx + c0, gy + c1, gx + c2, gy + c3
    w = a2 - a0
    h = a3 - a1
    cx = a0 + 0.5 * w
    cy = a1 + 0.5 * h
    dw = jnp.minimum(d2, BBOX_CLIP)
    dh = jnp.minimum(d3, BBOX_CLIP)
    pcx = d0 * w + cx
    pcy = d1 * h + cy
    pw = jnp.exp(dw) * w
    ph = jnp.exp(dh) * h
    x1 = jnp.clip(pcx - 0.5 * pw, 0.0, 512.0)
    y1 = jnp.clip(pcy - 0.5 * ph, 0.0, 512.0)
    x2 = jnp.clip(pcx + 0.5 * pw, 0.0, 512.0)
    y2 = jnp.clip(pcy + 0.5 * ph, 0.0, 512.0)
    sc = 1.0 / (1.0 + jnp.exp(-vals))
    valid = ((x2 - x1) >= MIN_SIZE) & ((y2 - y1) >= MIN_SIZE) & (sc > 0.0) \
            & (rank < 1000)
    return x1, y1, x2, y2, valid


def _nms_kernel(vrow_ref, irow_ref, icol_ref, drow_ref, dcol_ref, out_ref):
    f32 = jnp.float32
    irow = irow_ref[0]
    rank_r = jax.lax.broadcasted_iota(jnp.int32, (1, 1024), 1)
    x1r, y1r, x2r, y2r, valid_r = _decode_frame(
        irow, drow_ref[0, 0], drow_ref[0, 1], drow_ref[0, 2], drow_ref[0, 3],
        vrow_ref[0], rank_r)
    icol = icol_ref[0]
    rank_c = jax.lax.broadcasted_iota(jnp.int32, (1024, 1), 0)
    x1c, y1c, x2c, y2c, _ = _decode_frame(
        icol, dcol_ref[0, 0], dcol_ref[0, 1], dcol_ref[0, 2], dcol_ref[0, 3],
        jnp.zeros((1024, 1), f32), rank_c)
    area_r = (x2r - x1r) * (y2r - y1r)
    area_c = (x2c - x1c) * (y2c - y1c)

    lane64 = jax.lax.broadcasted_iota(jnp.int32, (1, 64), 1)
    valid_rf = valid_r.astype(f32)

    sup = [jnp.zeros((1, 64), f32) for _ in range(16)]
    for t in range(16):
        sl = slice(t * 64, (t + 1) * 64)
        ltx = jnp.maximum(x1c[sl], x1r)
        lty = jnp.maximum(y1c[sl], y1r)
        rbx = jnp.minimum(x2c[sl], x2r)
        rby = jnp.minimum(y2c[sl], y2r)
        wx = jnp.maximum(rbx - ltx, 0.0)
        wy = jnp.maximum(rby - lty, 0.0)
        inter = wx * wy
        iou = inter / (area_c[sl] + area_r - inter + 1e-9)
        Tt = (iou > NMS_THR).astype(f32)
        Dt = Tt[:, sl]
        vblk = valid_rf[:, sl]
        l = sup[t]
        for i2 in range(64):
            alive_i = (1.0 - l[:, i2:i2 + 1]) * vblk[:, i2:i2 + 1]
            upd = alive_i * Dt[i2:i2 + 1, :] * (lane64 > i2).astype(f32)
            l = jnp.maximum(l, upd)
        sup[t] = l
        alive = (1.0 - l) * vblk
        contrib = jnp.dot(alive, Tt, preferred_element_type=f32)
        for u in range(t + 1, 16):
            sup[u] = jnp.maximum(sup[u],
                                 jnp.minimum(contrib[:, u * 64:(u + 1) * 64],
                                             1.0))
    sup_row = jnp.concatenate(sup, axis=1)
    keep = (sup_row < 0.5) & valid_r
    kf = keep.astype(f32)
    dropf = (1.0 - kf) * (rank_r < 1000).astype(f32)

    iota_c = jax.lax.broadcasted_iota(jnp.int32, (1024, 1024), 0)
    iota_l = jax.lax.broadcasted_iota(jnp.int32, (1024, 1024), 1)
    U = (iota_c <= iota_l).astype(f32)
    pref_k = jnp.dot(kf, U, preferred_element_type=f32)
    pref_d = jnp.dot(dropf, U, preferred_element_type=f32)
    nk = jnp.sum(kf)
    pos2 = jnp.where(keep, pref_k - 1.0,
                     jnp.where(dropf > 0.5, nk + pref_d - 1.0,
                               rank_r.astype(f32)))
    P2 = (pos2 == iota_c.astype(f32)).astype(f32)
    hp = jax.lax.Precision.HIGHEST
    o0 = jax.lax.dot(P2, x1c, precision=hp)
    o1 = jax.lax.dot(P2, y1c, precision=hp)
    o2 = jax.lax.dot(P2, x2c, precision=hp)
    o3 = jax.lax.dot(P2, y2c, precision=hp)
    out_ref[0] = jnp.concatenate([o0, o1, o2, o3], axis=1)


def _nms(vrow, irow, icol, drow, dcol):
    B = vrow.shape[0]
    return pl.pallas_call(
        _nms_kernel,
        grid=(B,),
        in_specs=[
            pl.BlockSpec((1, 1, 1024), lambda i: (i, 0, 0)),
            pl.BlockSpec((1, 1, 1024), lambda i: (i, 0, 0)),
            pl.BlockSpec((1, 1024, 1), lambda i: (i, 0, 0)),
            pl.BlockSpec((1, 4, 1, 1024), lambda i: (i, 0, 0, 0)),
            pl.BlockSpec((1, 4, 1024, 1), lambda i: (i, 0, 0, 0)),
        ],
        out_specs=pl.BlockSpec((1, 1024, 4), lambda i: (i, 0, 0)),
        out_shape=jax.ShapeDtypeStruct((B, 1024, 4), jnp.float32),
    )(vrow, irow, icol, drow, dcol)


def kernel(images, features, w_conv, b_conv, w_cls, b_cls, w_bbox, b_bbox):
    B = features.shape[0]
    heads = _conv_heads(features, w_conv, b_conv, w_cls, b_cls, w_bbox, b_bbox)
    scores = heads[..., :10].reshape(B, 40960)
    deltas = heads[..., 10:50].reshape(B, 40960, 4)
    v, ix = _topk1024(scores)
    ixf = ix.reshape(B, 1024)
    dtop = jnp.take_along_axis(deltas, ixf[:, :, None], axis=1)
    vrow = v.reshape(B, 1, 1024)
    irow = ixf.reshape(B, 1, 1024)
    icol = ixf.reshape(B, 1024, 1)
    drow = jnp.transpose(dtop, (0, 2, 1))[:, :, None, :]
    dcol = jnp.transpose(dtop, (0, 2, 1))[:, :, :, None]
    out = _nms(vrow, irow, icol, drow, dcol)
    return out[:, :1000, :]


# speedup vs baseline: 13.3428x; 1.0892x over previous
import math

import numpy as np
import jax
import jax.numpy as jnp
from jax.experimental import pallas as pl
from jax.experimental.pallas import tpu as pltpu

SCALES = (8., 16., 32., 64., 128., 256., 512., 1024., 2048., 4096.)
RATIO = 0.125
NMS_THR = 0.7
MIN_SIZE = 1e-3
BBOX_CLIP = math.log(1000.0 / 16.0)


def _cell_anchors_np():
    scales = np.array(SCALES, dtype=np.float32)
    h_r = np.sqrt(np.array([RATIO], dtype=np.float32))
    w_r = (np.float32(1.0) / h_r).astype(np.float32)
    ws = (w_r[:, None] * scales[None, :]).reshape(-1).astype(np.float32)
    hs = (h_r[:, None] * scales[None, :]).reshape(-1).astype(np.float32)
    return np.round(np.stack([-ws, -hs, ws, hs], axis=1).astype(np.float32) / 2.0)

_CELL = _cell_anchors_np()


def _conv_kernel(xpad_ref, wtaps_ref, bconv_ref, whead_ref, bhead_ref, out_ref):
    lane = jax.lax.broadcasted_iota(jnp.int32, (4096, 1), 0) % 64
    acc = jnp.zeros((4096, 256), jnp.float32)
    for k in range(9):
        dy, dx = k // 3 - 1, k % 3 - 1
        off = 72 + dy * 64 + dx
        sl = xpad_ref[0, pl.ds(off, 4096), :]
        if dx == -1:
            sl = jnp.where(lane == 0, 0.0, sl)
        elif dx == 1:
            sl = jnp.where(lane == 63, 0.0, sl)
        acc = acc + jnp.dot(sl, wtaps_ref[k],
                            preferred_element_type=jnp.float32)
    t = jnp.maximum(acc + bconv_ref[0][None, :], 0.0)
    out_ref[0] = (jnp.dot(t, whead_ref[...], preferred_element_type=jnp.float32)
                  + bhead_ref[0][None, :])


def _conv_heads(features, w_conv, b_conv, w_cls, b_cls, w_bbox, b_bbox):
    B = features.shape[0]
    x = jnp.transpose(features, (0, 2, 3, 1)).reshape(B, 4096, 256)
    xpad = jnp.pad(x, ((0, 0), (72, 72), (0, 0)))
    wtaps = jnp.transpose(w_conv, (2, 3, 1, 0)).reshape(9, 256, 256)
    whead = jnp.concatenate([w_cls[:, :, 0, 0], w_bbox[:, :, 0, 0]], axis=0).T
    whead = jnp.pad(whead, ((0, 0), (0, 14)))
    bhead = jnp.pad(jnp.concatenate([b_cls, b_bbox]), (0, 14))[None, :]
    return pl.pallas_call(
        _conv_kernel,
        grid=(B,),
        in_specs=[
            pl.BlockSpec((1, 4240, 256), lambda i: (i, 0, 0)),
            pl.BlockSpec((9, 256, 256), lambda i: (0, 0, 0)),
            pl.BlockSpec((1, 256), lambda i: (0, 0)),
            pl.BlockSpec((256, 64), lambda i: (0, 0)),
            pl.BlockSpec((1, 64), lambda i: (0, 0)),
        ],
        out_specs=pl.BlockSpec((1, 4096, 64), lambda i: (i, 0, 0)),
        out_shape=jax.ShapeDtypeStruct((B, 4096, 64), jnp.float32),
    )(xpad, wtaps, b_conv[None, :], whead, bhead)


def _comp_gt(av, ai, bv, bi):
    return (av > bv) | ((av == bv) & (ai < bi))


def _flat_idx(shape):
    s = jax.lax.broadcasted_iota(jnp.int32, shape, len(shape) - 2)
    l = jax.lax.broadcasted_iota(jnp.int32, shape, len(shape) - 1)
    return s * 128 + l


def _xor_perm(x, j):
    if j < 128:
        l = jax.lax.broadcasted_iota(jnp.int32, x.shape, x.ndim - 1)
        n = x.shape[x.ndim - 1]
        lo = pltpu.roll(x, n - j, x.ndim - 1)
        hi = pltpu.roll(x, j, x.ndim - 1)
        return jnp.where((l & j) == 0, lo, hi)
    js = j // 128
    s = jax.lax.broadcasted_iota(jnp.int32, x.shape, x.ndim - 2)
    n = x.shape[x.ndim - 2]
    lo = pltpu.roll(x, n - js, x.ndim - 2)
    hi = pltpu.roll(x, js, x.ndim - 2)
    return jnp.where((s & js) == 0, lo, hi)


def _stage(v, ix, j, want_larger):
    pv = _xor_perm(v, j)
    pix = _xor_perm(ix, j)
    self_larger = _comp_gt(v, ix, pv, pix)
    sel = want_larger == self_larger
    return jnp.where(sel, v, pv), jnp.where(sel, ix, pix)


def _chunk_odd(shape):
    return (jax.lax.broadcasted_iota(jnp.int32, shape, 1) & 1) == 1


def _topk_kernel(s_ref, v_ref, ix_ref):
    v = s_ref[...]
    B = v.shape[0]
    ix = _flat_idx(v.shape) + 1024 * jax.lax.broadcasted_iota(
        jnp.int32, v.shape, 1)
    i = _flat_idx(v.shape)
    odd = _chunk_odd(v.shape)
    k = 2
    while k <= 1024:
        j = k // 2
        while j >= 1:
            wl = (((i & k) == 0) == ((i & j) == 0)) ^ odd
            v, ix = _stage(v, ix, j, wl)
            j //= 2
        k *= 2
    v = jnp.concatenate(
        [v, jnp.full((B, 24, 8, 128), -jnp.inf, jnp.float32)], axis=1)
    ix = jnp.concatenate([ix, jnp.zeros((B, 24, 8, 128), jnp.int32)], axis=1)
    m = 64
    while m > 1:
        v = v.reshape(B, m // 2, 2, 8, 128)
        ix = ix.reshape(B, m // 2, 2, 8, 128)
        av, bv, ai, bi = v[:, :, 0], v[:, :, 1], ix[:, :, 0], ix[:, :, 1]
        take = _comp_gt(av, ai, bv, bi)
        v = jnp.where(take, av, bv)
        ix = jnp.where(take, ai, bi)
        i = _flat_idx(v.shape)
        odd = _chunk_odd(v.shape)
        j = 512
        while j >= 1:
            wl = ((i & j) == 0) ^ odd
            v, ix = _stage(v, ix, j, wl)
            j //= 2
        m //= 2
    v_ref[...] = v[:, 0]
    ix_ref[...] = ix[:, 0]


def _topk1024(scores):
    B = scores.shape[0]
    return pl.pallas_call(
        _topk_kernel,
        out_shape=[jax.ShapeDtypeStruct((B, 8, 128), jnp.float32),
                   jax.ShapeDtypeStruct((B, 8, 128), jnp.int32)],
    )(scores.reshape(B, 40, 8, 128))


def _decode_frame(idx, d0, d1, d2, d3, vals, rank):
    a = idx % 10
    pix = idx // 10
    gy = (pix // 64).astype(jnp.float32) * 8.0
    gx = (pix % 64).astype(jnp.float32) * 8.0
    c0 = jnp.zeros_like(gx)
    c1 = jnp.zeros_like(gx)
    c2 = jnp.zeros_like(gx)
    c3 = jnp.zeros_like(gx)
    for av in range(10):
        m = a == av
        c0 = jnp.where(m, float(_CELL[av, 0]), c0)
        c1 = jnp.where(m, float(_CELL[av, 1]), c1)
        c2 = jnp.where(m, float(_CELL[av, 2]), c2)
        c3 = jnp.where(m, float(_CELL[av, 3]), c3)
    a0, a1, a2, a3 = gx + c0, gy + c1, gx + c2, gy + c3
    w = a2 - a0
    h = a3 - a1
    cx = a0 + 0.5 * w
    cy = a1 + 0.5 * h
    dw = jnp.minimum(d2, BBOX_CLIP)
    dh = jnp.minimum(d3, BBOX_CLIP)
    pcx = d0 * w + cx
    pcy = d1 * h + cy
    pw = jnp.exp(dw) * w
    ph = jnp.exp(dh) * h
    x1 = jnp.clip(pcx - 0.5 * pw, 0.0, 512.0)
    y1 = jnp.clip(pcy - 0.5 * ph, 0.0, 512.0)
    x2 = jnp.clip(pcx + 0.5 * pw, 0.0, 512.0)
    y2 = jnp.clip(pcy + 0.5 * ph, 0.0, 512.0)
    sc = 1.0 / (1.0 + jnp.exp(-vals))
    valid = ((x2 - x1) >= MIN_SIZE) & ((y2 - y1) >= MIN_SIZE) & (sc > 0.0) \
            & (rank < 1000)
    return x1, y1, x2, y2, valid


def _prefix1024(x_row):
    f32 = jnp.float32
    x8 = x_row.reshape(8, 128)
    lane = jax.lax.broadcasted_iota(jnp.int32, (8, 128), 1)
    p = x8
    d = 1
    while d < 128:
        p = p + jnp.where(lane >= d, pltpu.roll(p, d, 1), 0.0)
        d *= 2
    rs = p[:, 127:128]
    sub = jax.lax.broadcasted_iota(jnp.int32, (8, 1), 0)
    e = rs
    d = 1
    while d < 8:
        e = e + jnp.where(sub >= d, pltpu.roll(e, d, 0), 0.0)
        d *= 2
    out = p + (e - rs)
    return out.reshape(1, 1024).astype(f32)


def _nms_kernel(vrow_ref, irow_ref, icol_ref, drow_ref, dcol_ref, out_ref,
                dm_ref):
    f32 = jnp.float32
    irow = irow_ref[0]
    rank_r = jax.lax.broadcasted_iota(jnp.int32, (1, 1024), 1)
    x1r, y1r, x2r, y2r, valid_r = _decode_frame(
        irow, drow_ref[0, 0], drow_ref[0, 1], drow_ref[0, 2], drow_ref[0, 3],
        vrow_ref[0], rank_r)
    icol = icol_ref[0]
    rank_c = jax.lax.broadcasted_iota(jnp.int32, (1024, 1), 0)
    x1c, y1c, x2c, y2c, valid_c = _decode_frame(
        icol, dcol_ref[0, 0], dcol_ref[0, 1], dcol_ref[0, 2], dcol_ref[0, 3],
        jnp.zeros((1024, 1), f32), rank_c)
    area_r = (x2r - x1r) * (y2r - y1r)
    area_c = (x2c - x1c) * (y2c - y1c)
    valid_cf = valid_c.astype(f32)

    sub64 = jax.lax.broadcasted_iota(jnp.int32, (64, 64), 0)
    lane64b = jax.lax.broadcasted_iota(jnp.int32, (64, 64), 1)
    upper = (lane64b > sub64).astype(f32)
    valid_rf = valid_r.astype(f32)

    sup = [jnp.zeros((1, 64), f32) for _ in range(16)]
    for t in range(16):
        sl = slice(t * 64, (t + 1) * 64)
        ltx = jnp.maximum(x1c[sl], x1r)
        lty = jnp.maximum(y1c[sl], y1r)
        rbx = jnp.minimum(x2c[sl], x2r)
        rby = jnp.minimum(y2c[sl], y2r)
        wx = jnp.maximum(rbx - ltx, 0.0)
        wy = jnp.maximum(rby - lty, 0.0)
        inter = wx * wy
        iou = inter / (area_c[sl] + area_r - inter + 1e-9)
        Tt = (iou > NMS_THR).astype(f32)
        dm_ref[...] = Tt[:, sl] * valid_cf[sl] * upper
        vblk = valid_rf[:, sl]
        l = sup[t]
        for i2 in range(64):
            alive = l[:, i2:i2 + 1] == 0.0
            l = jnp.maximum(l, jnp.where(alive, dm_ref[i2:i2 + 1, :], 0.0))
        sup[t] = l
        alive_v = (1.0 - jnp.minimum(l, 1.0)) * vblk
        contrib = jnp.dot(alive_v, Tt, preferred_element_type=f32)
        for u in range(t + 1, 16):
            sup[u] = jnp.maximum(sup[u],
                                 jnp.minimum(contrib[:, u * 64:(u + 1) * 64],
                                             1.0))
    sup_row = jnp.concatenate(sup, axis=1)
    keep = (sup_row < 0.5) & valid_r
    kf = keep.astype(f32)
    dropf = (1.0 - kf) * (rank_r < 1000).astype(f32)

    pref_k = _prefix1024(kf)
    pref_d = _prefix1024(dropf)
    nk = jnp.sum(kf)
    pos2 = jnp.where(keep, pref_k - 1.0,
                     jnp.where(dropf > 0.5, nk + pref_d - 1.0,
                               rank_r.astype(f32)))
    iota_c = jax.lax.broadcasted_iota(jnp.int32, (1024, 1024), 0)
    P2 = (pos2 == iota_c.astype(f32)).astype(f32)
    X = jnp.concatenate([x1c, y1c, x2c, y2c], axis=1)
    out_ref[0] = jax.lax.dot(P2, X, precision=jax.lax.Precision.HIGHEST)


def _nms(vrow, irow, icol, drow, dcol):
    B = vrow.shape[0]
    return pl.pallas_call(
        _nms_kernel,
        grid=(B,),
        in_specs=[
            pl.BlockSpec((1, 1, 1024), lambda i: (i, 0, 0)),
            pl.BlockSpec((1, 1, 1024), lambda i: (i, 0, 0)),
            pl.BlockSpec((1, 1024, 1), lambda i: (i, 0, 0)),
            pl.BlockSpec((1, 4, 1, 1024), lambda i: (i, 0, 0, 0)),
            pl.BlockSpec((1, 4, 1024, 1), lambda i: (i, 0, 0, 0)),
        ],
        out_specs=pl.BlockSpec((1, 1024, 4), lambda i: (i, 0, 0)),
        out_shape=jax.ShapeDtypeStruct((B, 1024, 4), jnp.float32),
        scratch_shapes=[pltpu.VMEM((64, 64), jnp.float32)],
    )(vrow, irow, icol, drow, dcol)


def kernel(images, features, w_conv, b_conv, w_cls, b_cls, w_bbox, b_bbox):
    B = features.shape[0]
    heads = _conv_heads(features, w_conv, b_conv, w_cls, b_cls, w_bbox, b_bbox)
    scores = heads[..., :10].reshape(B, 40960)
    deltas = heads[..., 10:50].reshape(B, 40960, 4)
    v, ix = _topk1024(scores)
    ixf = ix.reshape(B, 1024)
    dtop = jnp.take_along_axis(deltas, ixf[:, :, None], axis=1)
    vrow = v.reshape(B, 1, 1024)
    irow = ixf.reshape(B, 1, 1024)
    icol = ixf.reshape(B, 1024, 1)
    drow = jnp.transpose(dtop, (0, 2, 1))[:, :, None, :]
    dcol = jnp.transpose(dtop, (0, 2, 1))[:, :, :, None]
    out = _nms(vrow, irow, icol, drow, dcol)
    return out[:, :1000, :]


# speedup vs baseline: 31.8698x; 2.3885x over previous
import math

import numpy as np
import jax
import jax.numpy as jnp
from jax.experimental import pallas as pl
from jax.experimental.pallas import tpu as pltpu

SCALES = (8., 16., 32., 64., 128., 256., 512., 1024., 2048., 4096.)
RATIO = 0.125
NMS_THR = 0.7
MIN_SIZE = 1e-3
BBOX_CLIP = math.log(1000.0 / 16.0)


def _cell_anchors_np():
    scales = np.array(SCALES, dtype=np.float32)
    h_r = np.sqrt(np.array([RATIO], dtype=np.float32))
    w_r = (np.float32(1.0) / h_r).astype(np.float32)
    ws = (w_r[:, None] * scales[None, :]).reshape(-1).astype(np.float32)
    hs = (h_r[:, None] * scales[None, :]).reshape(-1).astype(np.float32)
    return np.round(np.stack([-ws, -hs, ws, hs], axis=1).astype(np.float32) / 2.0)

_CELL = _cell_anchors_np()


def _conv_kernel(xpad_ref, wtaps_ref, bconv_ref, whead_ref, bhead_ref, out_ref):
    lane = jax.lax.broadcasted_iota(jnp.int32, (4096, 1), 0) % 64
    acc = jnp.zeros((4096, 256), jnp.float32)
    for k in range(9):
        dy, dx = k // 3 - 1, k % 3 - 1
        off = 72 + dy * 64 + dx
        sl = xpad_ref[0, pl.ds(off, 4096), :]
        if dx == -1:
            sl = jnp.where(lane == 0, 0.0, sl)
        elif dx == 1:
            sl = jnp.where(lane == 63, 0.0, sl)
        acc = acc + jnp.dot(sl, wtaps_ref[k],
                            preferred_element_type=jnp.float32)
    t = jnp.maximum(acc + bconv_ref[0][None, :], 0.0)
    out_ref[0] = (jnp.dot(t, whead_ref[...], preferred_element_type=jnp.float32)
                  + bhead_ref[0][None, :])


def _conv_heads(features, w_conv, b_conv, w_cls, b_cls, w_bbox, b_bbox):
    B = features.shape[0]
    x = jnp.transpose(features, (0, 2, 3, 1)).reshape(B, 4096, 256)
    xpad = jnp.pad(x, ((0, 0), (72, 72), (0, 0)))
    wtaps = jnp.transpose(w_conv, (2, 3, 1, 0)).reshape(9, 256, 256)
    whead = jnp.concatenate([w_cls[:, :, 0, 0], w_bbox[:, :, 0, 0]], axis=0).T
    whead = jnp.pad(whead, ((0, 0), (0, 14)))
    bhead = jnp.pad(jnp.concatenate([b_cls, b_bbox]), (0, 14))[None, :]
    return pl.pallas_call(
        _conv_kernel,
        grid=(B,),
        in_specs=[
            pl.BlockSpec((1, 4240, 256), lambda i: (i, 0, 0)),
            pl.BlockSpec((9, 256, 256), lambda i: (0, 0, 0)),
            pl.BlockSpec((1, 256), lambda i: (0, 0)),
            pl.BlockSpec((256, 64), lambda i: (0, 0)),
            pl.BlockSpec((1, 64), lambda i: (0, 0)),
        ],
        out_specs=pl.BlockSpec((1, 4096, 64), lambda i: (i, 0, 0)),
        out_shape=jax.ShapeDtypeStruct((B, 4096, 64), jnp.float32),
    )(xpad, wtaps, b_conv[None, :], whead, bhead)


def _comp_gt(av, ai, bv, bi):
    return (av > bv) | ((av == bv) & (ai < bi))


def _flat_idx(shape):
    s = jax.lax.broadcasted_iota(jnp.int32, shape, len(shape) - 2)
    l = jax.lax.broadcasted_iota(jnp.int32, shape, len(shape) - 1)
    return s * 128 + l


def _xor_perm(x, j):
    if j < 128:
        l = jax.lax.broadcasted_iota(jnp.int32, x.shape, x.ndim - 1)
        n = x.shape[x.ndim - 1]
        lo = pltpu.roll(x, n - j, x.ndim - 1)
        hi = pltpu.roll(x, j, x.ndim - 1)
        return jnp.where((l & j) == 0, lo, hi)
    js = j // 128
    s = jax.lax.broadcasted_iota(jnp.int32, x.shape, x.ndim - 2)
    n = x.shape[x.ndim - 2]
    lo = pltpu.roll(x, n - js, x.ndim - 2)
    hi = pltpu.roll(x, js, x.ndim - 2)
    return jnp.where((s & js) == 0, lo, hi)


def _stage(v, ix, j, want_larger):
    pv = _xor_perm(v, j)
    pix = _xor_perm(ix, j)
    self_larger = _comp_gt(v, ix, pv, pix)
    sel = want_larger == self_larger
    return jnp.where(sel, v, pv), jnp.where(sel, ix, pix)


def _chunk_odd(shape):
    return (jax.lax.broadcasted_iota(jnp.int32, shape, 1) & 1) == 1


def _topk_kernel(s_ref, v_ref, ix_ref):
    v = s_ref[...]
    B = v.shape[0]
    ix = _flat_idx(v.shape) + 1024 * jax.lax.broadcasted_iota(
        jnp.int32, v.shape, 1)
    i = _flat_idx(v.shape)
    odd = _chunk_odd(v.shape)
    k = 2
    while k <= 1024:
        j = k // 2
        while j >= 1:
            wl = (((i & k) == 0) == ((i & j) == 0)) ^ odd
            v, ix = _stage(v, ix, j, wl)
            j //= 2
        k *= 2
    v = jnp.concatenate(
        [v, jnp.full((B, 24, 8, 128), -jnp.inf, jnp.float32)], axis=1)
    ix = jnp.concatenate([ix, jnp.zeros((B, 24, 8, 128), jnp.int32)], axis=1)
    m = 64
    while m > 1:
        v = v.reshape(B, m // 2, 2, 8, 128)
        ix = ix.reshape(B, m // 2, 2, 8, 128)
        av, bv, ai, bi = v[:, :, 0], v[:, :, 1], ix[:, :, 0], ix[:, :, 1]
        take = _comp_gt(av, ai, bv, bi)
        v = jnp.where(take, av, bv)
        ix = jnp.where(take, ai, bi)
        i = _flat_idx(v.shape)
        odd = _chunk_odd(v.shape)
        j = 512
        while j >= 1:
            wl = ((i & j) == 0) ^ odd
            v, ix = _stage(v, ix, j, wl)
            j //= 2
        m //= 2
    v_ref[...] = v[:, 0]
    ix_ref[...] = ix[:, 0]


def _topk1024(scores):
    B = scores.shape[0]
    return pl.pallas_call(
        _topk_kernel,
        out_shape=[jax.ShapeDtypeStruct((B, 8, 128), jnp.float32),
                   jax.ShapeDtypeStruct((B, 8, 128), jnp.int32)],
    )(scores.reshape(B, 40, 8, 128))


def _decode_frame(idx, d0, d1, d2, d3, vals, rank):
    a = idx % 10
    pix = idx // 10
    gy = (pix // 64).astype(jnp.float32) * 8.0
    gx = (pix % 64).astype(jnp.float32) * 8.0
    c0 = jnp.zeros_like(gx)
    c1 = jnp.zeros_like(gx)
    c2 = jnp.zeros_like(gx)
    c3 = jnp.zeros_like(gx)
    for av in range(10):
        m = a == av
        c0 = jnp.where(m, float(_CELL[av, 0]), c0)
        c1 = jnp.where(m, float(_CELL[av, 1]), c1)
        c2 = jnp.where(m, float(_CELL[av, 2]), c2)
        c3 = jnp.where(m, float(_CELL[av, 3]), c3)
    a0, a1, a2, a3 = gx + c0, gy + c1, gx + c2, gy + c3
    w = a2 - a0
    h = a3 - a1
    cx = a0 + 0.5 * w
    cy = a1 + 0.5 * h
    dw = jnp.minimum(d2, BBOX_CLIP)
    dh = jnp.minimum(d3, BBOX_CLIP)
    pcx = d0 * w + cx
    pcy = d1 * h + cy
    pw = jnp.exp(dw) * w
    ph = jnp.exp(dh) * h
    x1 = jnp.clip(pcx - 0.5 * pw, 0.0, 512.0)
    y1 = jnp.clip(pcy - 0.5 * ph, 0.0, 512.0)
    x2 = jnp.clip(pcx + 0.5 * pw, 0.0, 512.0)
    y2 = jnp.clip(pcy + 0.5 * ph, 0.0, 512.0)
    sc = 1.0 / (1.0 + jnp.exp(-vals))
    valid = ((x2 - x1) >= MIN_SIZE) & ((y2 - y1) >= MIN_SIZE) & (sc > 0.0) \
            & (rank < 1000)
    return x1, y1, x2, y2, valid


def _prefix1024(x_row):
    f32 = jnp.float32
    x8 = x_row.reshape(8, 128)
    lane = jax.lax.broadcasted_iota(jnp.int32, (8, 128), 1)
    p = x8
    d = 1
    while d < 128:
        p = p + jnp.where(lane >= d, pltpu.roll(p, d, 1), 0.0)
        d *= 2
    rs = p[:, 127:128]
    sub = jax.lax.broadcasted_iota(jnp.int32, (8, 1), 0)
    e = rs
    d = 1
    while d < 8:
        e = e + jnp.where(sub >= d, pltpu.roll(e, d, 0), 0.0)
        d *= 2
    out = p + (e - rs)
    return out.reshape(1, 1024).astype(f32)


def _nms_kernel(vrow_ref, irow_ref, icol_ref, drow_ref, dcol_ref, out_ref,
                m_ref):
    f32 = jnp.float32
    irow = irow_ref[0]
    rank_r = jax.lax.broadcasted_iota(jnp.int32, (1, 1024), 1)
    x1r, y1r, x2r, y2r, valid_r = _decode_frame(
        irow, drow_ref[0, 0], drow_ref[0, 1], drow_ref[0, 2], drow_ref[0, 3],
        vrow_ref[0], rank_r)
    icol = icol_ref[0]
    rank_c = jax.lax.broadcasted_iota(jnp.int32, (1024, 1), 0)
    x1c, y1c, x2c, y2c, valid_c = _decode_frame(
        icol, dcol_ref[0, 0], dcol_ref[0, 1], dcol_ref[0, 2], dcol_ref[0, 3],
        jnp.zeros((1024, 1), f32), rank_c)
    area_r = (x2r - x1r) * (y2r - y1r)
    area_c = (x2c - x1c) * (y2c - y1c)
    valid_cf = valid_c.astype(f32)
    valid_rf = valid_r.astype(f32)

    lane1024 = jax.lax.broadcasted_iota(jnp.int32, (64, 1024), 1)
    for t in range(16):
        sl = slice(t * 64, (t + 1) * 64)
        ltx = jnp.maximum(x1c[sl], x1r)
        lty = jnp.maximum(y1c[sl], y1r)
        rbx = jnp.minimum(x2c[sl], x2r)
        rby = jnp.minimum(y2c[sl], y2r)
        wx = jnp.maximum(rbx - ltx, 0.0)
        wy = jnp.maximum(rby - lty, 0.0)
        inter = wx * wy
        iou = inter / (area_c[sl] + area_r - inter + 1e-9)
        gi = jax.lax.broadcasted_iota(jnp.int32, (64, 1024), 0) + t * 64
        m_ref[sl, :] = jnp.where((iou > NMS_THR) & (gi < lane1024),
                                 valid_cf[sl], 0.0)

    def cond(c):
        it, prev, kept = c
        return jnp.logical_and(it < 1025, jnp.any(prev != kept))

    def body(c):
        it, prev, kept = c
        sup = jnp.dot(kept, m_ref[...], preferred_element_type=f32)
        new = jnp.where(sup < 0.5, valid_rf, 0.0)
        return it + 1, kept, new

    _, _, kf = jax.lax.while_loop(
        cond, body, (jnp.int32(0), -jnp.ones((1, 1024), f32), valid_rf))
    keep = kf > 0.5
    dropf = (1.0 - kf) * (rank_r < 1000).astype(f32)

    pref_k = _prefix1024(kf)
    pref_d = _prefix1024(dropf)
    nk = jnp.sum(kf)
    pos2 = jnp.where(keep, pref_k - 1.0,
                     jnp.where(dropf > 0.5, nk + pref_d - 1.0,
                               rank_r.astype(f32)))
    iota_c = jax.lax.broadcasted_iota(jnp.int32, (1024, 1024), 0)
    P2 = (pos2 == iota_c.astype(f32)).astype(f32)
    X = jnp.concatenate([x1c, y1c, x2c, y2c], axis=1)
    out_ref[0] = jax.lax.dot(P2, X, precision=jax.lax.Precision.HIGHEST)


def _nms(vrow, irow, icol, drow, dcol):
    B = vrow.shape[0]
    return pl.pallas_call(
        _nms_kernel,
        grid=(B,),
        in_specs=[
            pl.BlockSpec((1, 1, 1024), lambda i: (i, 0, 0)),
            pl.BlockSpec((1, 1, 1024), lambda i: (i, 0, 0)),
            pl.BlockSpec((1, 1024, 1), lambda i: (i, 0, 0)),
            pl.BlockSpec((1, 4, 1, 1024), lambda i: (i, 0, 0, 0)),
            pl.BlockSpec((1, 4, 1024, 1), lambda i: (i, 0, 0, 0)),
        ],
        out_specs=pl.BlockSpec((1, 1024, 4), lambda i: (i, 0, 0)),
        out_shape=jax.ShapeDtypeStruct((B, 1024, 4), jnp.float32),
        scratch_shapes=[pltpu.VMEM((1024, 1024), jnp.float32)],
    )(vrow, irow, icol, drow, dcol)


def kernel(images, features, w_conv, b_conv, w_cls, b_cls, w_bbox, b_bbox):
    B = features.shape[0]
    heads = _conv_heads(features, w_conv, b_conv, w_cls, b_cls, w_bbox, b_bbox)
    scores = heads[..., :10].reshape(B, 40960)
    deltas = heads[..., 10:50].reshape(B, 40960, 4)
    v, ix = _topk1024(scores)
    ixf = ix.reshape(B, 1024)
    dtop = jnp.take_along_axis(deltas, ixf[:, :, None], axis=1)
    vrow = v.reshape(B, 1, 1024)
    irow = ixf.reshape(B, 1, 1024)
    icol = ixf.reshape(B, 1024, 1)
    drow = jnp.transpose(dtop, (0, 2, 1))[:, :, None, :]
    dcol = jnp.transpose(dtop, (0, 2, 1))[:, :, :, None]
    out = _nms(vrow, irow, icol, drow, dcol)
    return out[:, :1000, :]


# speedup vs baseline: 37.6378x; 1.1810x over previous
import math

import numpy as np
import jax
import jax.numpy as jnp
from jax.experimental import pallas as pl
from jax.experimental.pallas import tpu as pltpu

SCALES = (8., 16., 32., 64., 128., 256., 512., 1024., 2048., 4096.)
RATIO = 0.125
NMS_THR = 0.7
MIN_SIZE = 1e-3
BBOX_CLIP = math.log(1000.0 / 16.0)


def _cell_anchors_np():
    scales = np.array(SCALES, dtype=np.float32)
    h_r = np.sqrt(np.array([RATIO], dtype=np.float32))
    w_r = (np.float32(1.0) / h_r).astype(np.float32)
    ws = (w_r[:, None] * scales[None, :]).reshape(-1).astype(np.float32)
    hs = (h_r[:, None] * scales[None, :]).reshape(-1).astype(np.float32)
    return np.round(np.stack([-ws, -hs, ws, hs], axis=1).astype(np.float32) / 2.0)

_CELL = _cell_anchors_np()


def _conv_kernel(x_ref, wtaps_ref, bconv_ref, whead_ref, bhead_ref, out_ref):
    lane = jax.lax.broadcasted_iota(jnp.int32, (1, 4096), 1)
    x = lane % 64
    y = lane // 64
    X = x_ref[0]
    acc = jnp.zeros((256, 4096), jnp.float32)
    for k in range(9):
        dy, dx = k // 3 - 1, k % 3 - 1
        s = dy * 64 + dx
        xs = pltpu.roll(X, (4096 - s) % 4096, 1) if s else X
        m = None
        if dx == -1:
            m = x > 0
        elif dx == 1:
            m = x < 63
        if dy == -1:
            my = y > 0
            m = my if m is None else (m & my)
        elif dy == 1:
            my = y < 63
            m = my if m is None else (m & my)
        if m is not None:
            xs = jnp.where(m, xs, 0.0)
        acc = acc + jnp.dot(wtaps_ref[k], xs,
                            preferred_element_type=jnp.float32)
    t = jnp.maximum(acc + jnp.transpose(bconv_ref[...], (1, 0)), 0.0)
    out_ref[0] = (jnp.dot(whead_ref[...], t, preferred_element_type=jnp.float32)
                  + jnp.transpose(bhead_ref[...], (1, 0)))


def _conv_heads(features, w_conv, b_conv, w_cls, b_cls, w_bbox, b_bbox):
    B = features.shape[0]
    x = features.reshape(B, 256, 4096)
    wtaps = jnp.transpose(w_conv.reshape(256, 256, 9), (2, 0, 1))
    whead = jnp.concatenate([w_cls[:, :, 0, 0], w_bbox[:, :, 0, 0],
                             jnp.zeros((14, 256), jnp.float32)], axis=0)
    bhead = jnp.pad(jnp.concatenate([b_cls, b_bbox]), (0, 14))[None, :]
    return pl.pallas_call(
        _conv_kernel,
        grid=(B,),
        in_specs=[
            pl.BlockSpec((1, 256, 4096), lambda i: (i, 0, 0)),
            pl.BlockSpec((9, 256, 256), lambda i: (0, 0, 0)),
            pl.BlockSpec((1, 256), lambda i: (0, 0)),
            pl.BlockSpec((64, 256), lambda i: (0, 0)),
            pl.BlockSpec((1, 64), lambda i: (0, 0)),
        ],
        out_specs=pl.BlockSpec((1, 64, 4096), lambda i: (i, 0, 0)),
        out_shape=jax.ShapeDtypeStruct((B, 64, 4096), jnp.float32),
    )(x, wtaps, b_conv[None, :], whead, bhead)


def _comp_gt(av, ai, bv, bi):
    return (av > bv) | ((av == bv) & (ai < bi))


def _flat_idx(shape):
    s = jax.lax.broadcasted_iota(jnp.int32, shape, len(shape) - 2)
    l = jax.lax.broadcasted_iota(jnp.int32, shape, len(shape) - 1)
    return s * 128 + l


def _xor_perm(x, j):
    if j < 128:
        l = jax.lax.broadcasted_iota(jnp.int32, x.shape, x.ndim - 1)
        n = x.shape[x.ndim - 1]
        lo = pltpu.roll(x, n - j, x.ndim - 1)
        hi = pltpu.roll(x, j, x.ndim - 1)
        return jnp.where((l & j) == 0, lo, hi)
    js = j // 128
    s = jax.lax.broadcasted_iota(jnp.int32, x.shape, x.ndim - 2)
    n = x.shape[x.ndim - 2]
    lo = pltpu.roll(x, n - js, x.ndim - 2)
    hi = pltpu.roll(x, js, x.ndim - 2)
    return jnp.where((s & js) == 0, lo, hi)


def _stage(v, ix, j, want_larger):
    pv = _xor_perm(v, j)
    pix = _xor_perm(ix, j)
    self_larger = _comp_gt(v, ix, pv, pix)
    sel = want_larger == self_larger
    return jnp.where(sel, v, pv), jnp.where(sel, ix, pix)


def _chunk_odd(shape):
    return (jax.lax.broadcasted_iota(jnp.int32, shape, 1) & 1) == 1


def _topk_kernel(s_ref, v_ref, ix_ref):
    v = s_ref[...].reshape(s_ref.shape[0], 40, 8, 128)
    B = v.shape[0]
    pos = _flat_idx(v.shape) + 1024 * jax.lax.broadcasted_iota(
        jnp.int32, v.shape, 1)
    ix = (pos & 4095) * 10 + (pos >> 12)
    i = _flat_idx(v.shape)
    odd = _chunk_odd(v.shape)
    k = 2
    while k <= 1024:
        j = k // 2
        while j >= 1:
            wl = (((i & k) == 0) == ((i & j) == 0)) ^ odd
            v, ix = _stage(v, ix, j, wl)
            j //= 2
        k *= 2
    v = jnp.concatenate(
        [v, jnp.full((B, 24, 8, 128), -jnp.inf, jnp.float32)], axis=1)
    ix = jnp.concatenate([ix, jnp.zeros((B, 24, 8, 128), jnp.int32)], axis=1)
    m = 64
    while m > 1:
        v = v.reshape(B, m // 2, 2, 8, 128)
        ix = ix.reshape(B, m // 2, 2, 8, 128)
        av, bv, ai, bi = v[:, :, 0], v[:, :, 1], ix[:, :, 0], ix[:, :, 1]
        take = _comp_gt(av, ai, bv, bi)
        v = jnp.where(take, av, bv)
        ix = jnp.where(take, ai, bi)
        i = _flat_idx(v.shape)
        odd = _chunk_odd(v.shape)
        j = 512
        while j >= 1:
            wl = ((i & j) == 0) ^ odd
            v, ix = _stage(v, ix, j, wl)
            j //= 2
        m //= 2
    v_ref[...] = v[:, 0]
    ix_ref[...] = ix[:, 0]


def _topk1024(heads):
    B = heads.shape[0]
    return pl.pallas_call(
        _topk_kernel,
        grid=(1,),
        in_specs=[pl.BlockSpec((B, 320, 128), lambda i: (0, 0, 0))],
        out_specs=[pl.BlockSpec((B, 8, 128), lambda i: (0, 0, 0)),
                   pl.BlockSpec((B, 8, 128), lambda i: (0, 0, 0))],
        out_shape=[jax.ShapeDtypeStruct((B, 8, 128), jnp.float32),
                   jax.ShapeDtypeStruct((B, 8, 128), jnp.int32)],
    )(heads.reshape(B, 2048, 128))


def _decode_frame(idx, d0, d1, d2, d3, vals, rank):
    a = idx % 10
    pix = idx // 10
    gy = (pix // 64).astype(jnp.float32) * 8.0
    gx = (pix % 64).astype(jnp.float32) * 8.0
    c0 = jnp.zeros_like(gx)
    c1 = jnp.zeros_like(gx)
    c2 = jnp.zeros_like(gx)
    c3 = jnp.zeros_like(gx)
    for av in range(10):
        m = a == av
        c0 = jnp.where(m, float(_CELL[av, 0]), c0)
        c1 = jnp.where(m, float(_CELL[av, 1]), c1)
        c2 = jnp.where(m, float(_CELL[av, 2]), c2)
        c3 = jnp.where(m, float(_CELL[av, 3]), c3)
    a0, a1, a2, a3 = gx + c0, gy + c1, gx + c2, gy + c3
    w = a2 - a0
    h = a3 - a1
    cx = a0 + 0.5 * w
    cy = a1 + 0.5 * h
    dw = jnp.minimum(d2, BBOX_CLIP)
    dh = jnp.minimum(d3, BBOX_CLIP)
    pcx = d0 * w + cx
    pcy = d1 * h + cy
    pw = jnp.exp(dw) * w
    ph = jnp.exp(dh) * h
    x1 = jnp.clip(pcx - 0.5 * pw, 0.0, 512.0)
    y1 = jnp.clip(pcy - 0.5 * ph, 0.0, 512.0)
    x2 = jnp.clip(pcx + 0.5 * pw, 0.0, 512.0)
    y2 = jnp.clip(pcy + 0.5 * ph, 0.0, 512.0)
    sc = 1.0 / (1.0 + jnp.exp(-vals))
    valid = ((x2 - x1) >= MIN_SIZE) & ((y2 - y1) >= MIN_SIZE) & (sc > 0.0) \
            & (rank < 1000)
    return x1, y1, x2, y2, valid


def _prefix1024(x_row):
    f32 = jnp.float32
    x8 = x_row.reshape(8, 128)
    lane = jax.lax.broadcasted_iota(jnp.int32, (8, 128), 1)
    p = x8
    d = 1
    while d < 128:
        p = p + jnp.where(lane >= d, pltpu.roll(p, d, 1), 0.0)
        d *= 2
    rs = p[:, 127:128]
    sub = jax.lax.broadcasted_iota(jnp.int32, (8, 1), 0)
    e = rs
    d = 1
    while d < 8:
        e = e + jnp.where(sub >= d, pltpu.roll(e, d, 0), 0.0)
        d *= 2
    out = p + (e - rs)
    return out.reshape(1, 1024).astype(f32)


def _nms_kernel(vrow_ref, irow_ref, icol_ref, drow_ref, dcol_ref, out_ref,
                m_ref):
    f32 = jnp.float32
    irow = irow_ref[0]
    rank_r = jax.lax.broadcasted_iota(jnp.int32, (1, 1024), 1)
    x1r, y1r, x2r, y2r, valid_r = _decode_frame(
        irow, drow_ref[0, 0], drow_ref[0, 1], drow_ref[0, 2], drow_ref[0, 3],
        vrow_ref[0], rank_r)
    icol = icol_ref[0]
    rank_c = jax.lax.broadcasted_iota(jnp.int32, (1024, 1), 0)
    x1c, y1c, x2c, y2c, valid_c = _decode_frame(
        icol, dcol_ref[0, 0], dcol_ref[0, 1], dcol_ref[0, 2], dcol_ref[0, 3],
        jnp.zeros((1024, 1), f32), rank_c)
    area_r = (x2r - x1r) * (y2r - y1r)
    area_c = (x2c - x1c) * (y2c - y1c)
    valid_cf = valid_c.astype(f32)
    valid_rf = valid_r.astype(f32)

    lane1024 = jax.lax.broadcasted_iota(jnp.int32, (64, 1024), 1)
    for t in range(16):
        sl = slice(t * 64, (t + 1) * 64)
        ltx = jnp.maximum(x1c[sl], x1r)
        lty = jnp.maximum(y1c[sl], y1r)
        rbx = jnp.minimum(x2c[sl], x2r)
        rby = jnp.minimum(y2c[sl], y2r)
        wx = jnp.maximum(rbx - ltx, 0.0)
        wy = jnp.maximum(rby - lty, 0.0)
        inter = wx * wy
        iou = inter / (area_c[sl] + area_r - inter + 1e-9)
        gi = jax.lax.broadcasted_iota(jnp.int32, (64, 1024), 0) + t * 64
        m_ref[sl, :] = jnp.where((iou > NMS_THR) & (gi < lane1024),
                                 valid_cf[sl], 0.0)

    def cond(c):
        it, prev, kept = c
        return jnp.logical_and(it < 1025, jnp.any(prev != kept))

    def body(c):
        it, prev, kept = c
        sup = jnp.dot(kept, m_ref[...], preferred_element_type=f32)
        new = jnp.where(sup < 0.5, valid_rf, 0.0)
        return it + 1, kept, new

    _, _, kf = jax.lax.while_loop(
        cond, body, (jnp.int32(0), -jnp.ones((1, 1024), f32), valid_rf))
    keep = kf > 0.5
    dropf = (1.0 - kf) * (rank_r < 1000).astype(f32)

    pref_k = _prefix1024(kf)
    pref_d = _prefix1024(dropf)
    nk = jnp.sum(kf)
    pos2 = jnp.where(keep, pref_k - 1.0,
                     jnp.where(dropf > 0.5, nk + pref_d - 1.0,
                               rank_r.astype(f32)))
    iota_c = jax.lax.broadcasted_iota(jnp.int32, (1024, 1024), 0)
    P2 = (pos2 == iota_c.astype(f32)).astype(f32)
    X = jnp.concatenate([x1c, y1c, x2c, y2c], axis=1)
    out_ref[0] = jax.lax.dot(P2, X, precision=jax.lax.Precision.HIGHEST)


def _nms(vrow, irow, icol, drow, dcol):
    B = vrow.shape[0]
    return pl.pallas_call(
        _nms_kernel,
        grid=(B,),
        in_specs=[
            pl.BlockSpec((1, 1, 1024), lambda i: (i, 0, 0)),
            pl.BlockSpec((1, 1, 1024), lambda i: (i, 0, 0)),
            pl.BlockSpec((1, 1024, 1), lambda i: (i, 0, 0)),
            pl.BlockSpec((1, 4, 1, 1024), lambda i: (i, 0, 0, 0)),
            pl.BlockSpec((1, 4, 1024, 1), lambda i: (i, 0, 0, 0)),
        ],
        out_specs=pl.BlockSpec((1, 1024, 4), lambda i: (i, 0, 0)),
        out_shape=jax.ShapeDtypeStruct((B, 1024, 4), jnp.float32),
        scratch_shapes=[pltpu.VMEM((1024, 1024), jnp.float32)],
    )(vrow, irow, icol, drow, dcol)


def kernel(images, features, w_conv, b_conv, w_cls, b_cls, w_bbox, b_bbox):
    B = features.shape[0]
    heads = _conv_heads(features, w_conv, b_conv, w_cls, b_cls, w_bbox, b_bbox)
    v, ix = _topk1024(heads)
    ixf = ix.reshape(B, 1024)
    a = ixf % 10
    pix = ixf // 10
    gidx = ((10 + a[:, None, :] * 4
             + jnp.arange(4, dtype=jnp.int32)[None, :, None]) * 4096
            + pix[:, None, :])
    g = jnp.take_along_axis(heads.reshape(B, 262144),
                            gidx.reshape(B, 4096), axis=1).reshape(B, 4, 1024)
    vrow = v.reshape(B, 1, 1024)
    irow = ixf.reshape(B, 1, 1024)
    icol = ixf.reshape(B, 1024, 1)
    drow = g[:, :, None, :]
    dcol = g[:, :, :, None]
    out = _nms(vrow, irow, icol, drow, dcol)
    return out[:, :1000, :]
